# Initial kernel scaffold; baseline (speedup 1.0000x reference)
import functools

import jax
import jax.numpy as jnp
from jax import lax
from jax.experimental import pallas as pl
from jax.experimental.pallas import tpu as pltpu
from jax.experimental.pallas import tpu_sc as plsc

NW = 32
NS = 16
CHUNK = 128
ZROWS = 64
DEGW = 16


def _sc_aggregate(x, src3, dst3, n_pad, nj):
    n, f = x.shape
    rows_out = n // NS
    zch = n_pad // (NS * ZROWS)

    mesh = plsc.VectorSubcoreMesh(core_axis_name="c", subcore_axis_name="s")

    @functools.partial(
        pl.kernel,
        mesh=mesh,
        out_type=[
            jax.ShapeDtypeStruct((2, n, f), jnp.float32),
            jax.ShapeDtypeStruct((2, n, DEGW), jnp.float32),
        ],
        scratch_types=[
            pltpu.VMEM((nj, CHUNK), jnp.int32),
            pltpu.VMEM((nj, CHUNK), jnp.int32),
            pltpu.VMEM((CHUNK, f), jnp.float32),
            pltpu.VMEM((CHUNK, DEGW), jnp.float32),
            pltpu.VMEM((ZROWS, f), jnp.float32),
            pltpu.VMEM((ZROWS, DEGW), jnp.float32),
            pltpu.VMEM_SHARED((n_pad, f), jnp.float32),
            pltpu.VMEM_SHARED((n_pad, DEGW), jnp.float32),
        ],
    )
    def agg_kernel(x_hbm, src_hbm, dst_hbm, agg_out, deg_out,
                   src_v, dst_v, rows_v, ones_v, zb_v, zbd_v, agg_sh, deg_sh):
        c = lax.axis_index("c")
        s = lax.axis_index("s")
        wid = s * 2 + c

        def fill_row(i, carry):
            for k in range(f // 16):
                zb_v[i, pl.ds(k * 16, 16)] = jnp.zeros((16,), jnp.float32)
            zbd_v[i, :] = jnp.zeros((16,), jnp.float32)
            return carry
        lax.fori_loop(0, ZROWS, fill_row, 0)

        def fill_ones(i, carry):
            ones_v[i, :] = jnp.ones((16,), jnp.float32)
            return carry
        lax.fori_loop(0, CHUNK, fill_ones, 0)

        def zero_chunk(t, carry):
            row0 = s * (zch * ZROWS) + t * ZROWS
            pltpu.sync_copy(zb_v, agg_sh.at[pl.ds(row0, ZROWS)])
            pltpu.sync_copy(zbd_v, deg_sh.at[pl.ds(row0, ZROWS)])
            return carry
        lax.fori_loop(0, zch, zero_chunk, 0)

        plsc.subcore_barrier()

        pltpu.sync_copy(src_hbm.at[wid], src_v)
        pltpu.sync_copy(dst_hbm.at[wid], dst_v)

        def do_chunk(j, carry):
            pltpu.sync_copy(x_hbm.at[src_v.at[j]], rows_v)
            pltpu.sync_copy(rows_v, agg_sh.at[dst_v.at[j]], add=True)
            pltpu.sync_copy(ones_v, deg_sh.at[dst_v.at[j]], add=True)
            return carry
        lax.fori_loop(0, nj, do_chunk, 0)

        plsc.subcore_barrier()

        row0 = s * rows_out
        pltpu.sync_copy(agg_sh.at[pl.ds(row0, rows_out)],
                        agg_out.at[c, pl.ds(row0, rows_out)])
        pltpu.sync_copy(deg_sh.at[pl.ds(row0, rows_out)],
                        deg_out.at[c, pl.ds(row0, rows_out)])

    return agg_kernel(x, src3, dst3)


def _mlp(x, agg2, deg2, W1, b1, W2, b2, W3, b3):
    n, f = x.shape
    h1 = W1.shape[1]
    h2 = W2.shape[1]
    cc = W3.shape[1]
    bm = 2000
    grid = (n // bm,)

    def body(x_ref, a_ref, d_ref, w1_ref, b1_ref, w2_ref, b2_ref,
             w3_ref, b3_ref, o_ref):
        xb = x_ref[...]
        a = a_ref[0] + a_ref[1]
        d = d_ref[0, :, 0:1] + d_ref[1, :, 0:1]
        a = a / jnp.maximum(d, 1.0)
        w1 = w1_ref[...]
        h = (jnp.dot(xb, w1[0:f], preferred_element_type=jnp.float32)
             + jnp.dot(a, w1[f:2 * f], preferred_element_type=jnp.float32)
             + b1_ref[...])
        h = jnp.maximum(h, 0.0)
        h = jnp.dot(h, w2_ref[...], preferred_element_type=jnp.float32) + b2_ref[...]
        h = jnp.maximum(h, 0.0)
        o_ref[...] = (jnp.dot(h, w3_ref[...], preferred_element_type=jnp.float32)
                      + b3_ref[...])

    return pl.pallas_call(
        body,
        grid=grid,
        in_specs=[
            pl.BlockSpec((bm, f), lambda i: (i, 0)),
            pl.BlockSpec((2, bm, f), lambda i: (0, i, 0)),
            pl.BlockSpec((2, bm, DEGW), lambda i: (0, i, 0)),
            pl.BlockSpec((2 * f, h1), lambda i: (0, 0)),
            pl.BlockSpec((1, h1), lambda i: (0, 0)),
            pl.BlockSpec((h1, h2), lambda i: (0, 0)),
            pl.BlockSpec((1, h2), lambda i: (0, 0)),
            pl.BlockSpec((h2, cc), lambda i: (0, 0)),
            pl.BlockSpec((1, cc), lambda i: (0, 0)),
        ],
        out_specs=pl.BlockSpec((bm, cc), lambda i: (i, 0)),
        out_shape=jax.ShapeDtypeStruct((n, cc), jnp.float32),
    )(x, agg2, deg2, W1, b1, W2, b2, W3, b3)


def kernel(inputs, neighbor, W1, b1, W2, b2, W3, b3):
    x = inputs[:, 0, :]
    n = x.shape[0]
    src = neighbor[0]
    dst = neighbor[1]
    e = src.shape[0]

    nj = -(-e // (NW * CHUNK))
    e_pad = NW * nj * CHUNK
    if e_pad > e:
        src = jnp.concatenate([src, jnp.zeros((e_pad - e,), jnp.int32)])
        dst = jnp.concatenate([dst, jnp.full((e_pad - e,), n, jnp.int32)])
    src3 = src.reshape(NW, nj, CHUNK)
    dst3 = dst.reshape(NW, nj, CHUNK)

    n_pad = -(-(n + 1) // (NS * ZROWS)) * (NS * ZROWS)

    agg2, deg2 = _sc_aggregate(x, src3, dst3, n_pad, nj)
    return _mlp(x, agg2, deg2, W1, b1.reshape(1, -1), W2, b2.reshape(1, -1),
                W3, b3.reshape(1, -1))


# speedup vs baseline: 6.8604x; 6.8604x over previous
import functools

import jax
import jax.numpy as jnp
from jax import lax
from jax.experimental import pallas as pl
from jax.experimental.pallas import tpu as pltpu
from jax.experimental.pallas import tpu_sc as plsc

NS = 16
CHUNK = 128
ZROWS = 64
DEGW = 16


def _sc_aggregate(xa, xb, src3, dst3, n_pad, nj):
    n, fh = xa.shape
    rows_out = n_pad // NS
    zch = n_pad // (NS * ZROWS)

    mesh = plsc.VectorSubcoreMesh(core_axis_name="c", subcore_axis_name="s")

    @functools.partial(
        pl.kernel,
        mesh=mesh,
        compiler_params=pltpu.CompilerParams(use_tc_tiling_on_sc=False),
        out_type=[
            jax.ShapeDtypeStruct((2, n_pad, fh), jnp.float32),
            jax.ShapeDtypeStruct((n_pad, DEGW), jnp.float32),
        ],
        scratch_types=[
            pltpu.VMEM((nj, CHUNK), jnp.int32),
            pltpu.VMEM((nj, CHUNK), jnp.int32),
            pltpu.VMEM((CHUNK, fh), jnp.float32),
            pltpu.VMEM((CHUNK, DEGW), jnp.float32),
            pltpu.VMEM((ZROWS, fh), jnp.float32),
            pltpu.VMEM((ZROWS, DEGW), jnp.float32),
            pltpu.VMEM_SHARED((n_pad, fh), jnp.float32),
            pltpu.VMEM_SHARED((n_pad, DEGW), jnp.float32),
        ],
    )
    def agg_kernel(xa_hbm, xb_hbm, src_hbm, dst_hbm, agg_out, deg_out,
                   src_v, dst_v, rows_v, ones_v, zb_v, zbd_v, agg_sh, deg_sh):
        c = lax.axis_index("c")
        s = lax.axis_index("s")

        def fill_row(i, carry):
            for k in range(fh // 16):
                zb_v[i, pl.ds(k * 16, 16)] = jnp.zeros((16,), jnp.float32)
            zbd_v[i, :] = jnp.zeros((16,), jnp.float32)
            return carry
        lax.fori_loop(0, ZROWS, fill_row, 0)

        def fill_ones(i, carry):
            ones_v[i, :] = jnp.ones((16,), jnp.float32)
            return carry
        lax.fori_loop(0, CHUNK, fill_ones, 0)

        def zero_chunk(t, carry):
            row0 = s * (zch * ZROWS) + t * ZROWS
            pltpu.sync_copy(zb_v, agg_sh.at[pl.ds(row0, ZROWS)])
            pltpu.sync_copy(zbd_v, deg_sh.at[pl.ds(row0, ZROWS)])
            return carry
        lax.fori_loop(0, zch, zero_chunk, 0)

        plsc.subcore_barrier()

        pltpu.sync_copy(src_hbm.at[s], src_v)
        pltpu.sync_copy(dst_hbm.at[s], dst_v)

        def do_chunk(j, carry):
            @pl.when(c == 0)
            def _():
                pltpu.sync_copy(xa_hbm.at[src_v.at[j]], rows_v)
                pltpu.sync_copy(ones_v, deg_sh.at[dst_v.at[j]], add=True)

            @pl.when(c == 1)
            def _():
                pltpu.sync_copy(xb_hbm.at[src_v.at[j]], rows_v)

            pltpu.sync_copy(rows_v, agg_sh.at[dst_v.at[j]], add=True)
            return carry
        lax.fori_loop(0, nj, do_chunk, 0)

        plsc.subcore_barrier()

        row0 = s * rows_out
        pltpu.sync_copy(agg_sh.at[pl.ds(row0, rows_out)],
                        agg_out.at[c, pl.ds(row0, rows_out)])

        @pl.when(c == 0)
        def _():
            pltpu.sync_copy(deg_sh.at[pl.ds(row0, rows_out)],
                            deg_out.at[pl.ds(row0, rows_out)])

    return agg_kernel(xa, xb, src3, dst3)


def _mlp(x, agg2, deg, W1, b1, W2, b2, W3, b3):
    n, f = x.shape
    fh = f // 2
    h1 = W1.shape[1]
    h2 = W2.shape[1]
    cc = W3.shape[1]
    bm = 2000
    grid = (n // bm,)

    def body(x_ref, a_ref, d_ref, w1_ref, b1_ref, w2_ref, b2_ref,
             w3_ref, b3_ref, o_ref):
        xb = x_ref[...]
        a = jnp.concatenate([a_ref[0], a_ref[1]], axis=-1)
        d = d_ref[:, 0:1]
        a = a / jnp.maximum(d, 1.0)
        w1 = w1_ref[...]
        h = (jnp.dot(xb, w1[0:f], preferred_element_type=jnp.float32)
             + jnp.dot(a, w1[f:2 * f], preferred_element_type=jnp.float32)
             + b1_ref[...])
        h = jnp.maximum(h, 0.0)
        h = jnp.dot(h, w2_ref[...], preferred_element_type=jnp.float32) + b2_ref[...]
        h = jnp.maximum(h, 0.0)
        o_ref[...] = (jnp.dot(h, w3_ref[...], preferred_element_type=jnp.float32)
                      + b3_ref[...])

    return pl.pallas_call(
        body,
        grid=grid,
        in_specs=[
            pl.BlockSpec((bm, f), lambda i: (i, 0)),
            pl.BlockSpec((2, bm, fh), lambda i: (0, i, 0)),
            pl.BlockSpec((bm, DEGW), lambda i: (i, 0)),
            pl.BlockSpec((2 * f, h1), lambda i: (0, 0)),
            pl.BlockSpec((1, h1), lambda i: (0, 0)),
            pl.BlockSpec((h1, h2), lambda i: (0, 0)),
            pl.BlockSpec((1, h2), lambda i: (0, 0)),
            pl.BlockSpec((h2, cc), lambda i: (0, 0)),
            pl.BlockSpec((1, cc), lambda i: (0, 0)),
        ],
        out_specs=pl.BlockSpec((bm, cc), lambda i: (i, 0)),
        out_shape=jax.ShapeDtypeStruct((n, cc), jnp.float32),
    )(x, agg2, deg, W1, b1, W2, b2, W3, b3)


def kernel(inputs, neighbor, W1, b1, W2, b2, W3, b3):
    x = inputs[:, 0, :]
    n, f = x.shape
    fh = f // 2
    xa = x[:, :fh]
    xb = x[:, fh:]
    src = neighbor[0]
    dst = neighbor[1]
    e = src.shape[0]

    nj = -(-e // (NS * CHUNK))
    e_pad = NS * nj * CHUNK
    if e_pad > e:
        src = jnp.concatenate([src, jnp.zeros((e_pad - e,), jnp.int32)])
        dst = jnp.concatenate([dst, jnp.full((e_pad - e,), n, jnp.int32)])
    src3 = src.reshape(NS, nj, CHUNK)
    dst3 = dst.reshape(NS, nj, CHUNK)

    n_pad = -(-(n + 1) // (NS * ZROWS)) * (NS * ZROWS)

    agg2, deg = _sc_aggregate(xa, xb, src3, dst3, n_pad, nj)
    return _mlp(x, agg2, deg, W1, b1.reshape(1, -1), W2, b2.reshape(1, -1),
                W3, b3.reshape(1, -1))
